# baseline (speedup 1.0000x reference)
import jax
import jax.numpy as jnp
from jax.experimental import pallas as pl
from jax.experimental.pallas import tpu as pltpu

_T, _D, _E, _F, _K = 2048, 1024, 8, 1024, 4
_BT = 1024
_EP = 2


def _moe_body(x_ref, wg_ref, w1_ref, b1_ref, w2_ref, b2_ref, out_ref,
              g_ref, c_ref):
    e = pl.program_id(1)

    @pl.when(e == 1)
    def _():
        out_ref[...] = c_ref[...]

    @pl.when(e > 1)
    def _():
        out_ref[...] += c_ref[...]

    @pl.when(e == 0)
    def _():
        logits = jax.lax.dot_general(
            x_ref[...].astype(jnp.bfloat16), wg_ref[...], (((1,), (0,)), ((), ())),
            preferred_element_type=jnp.float32)
        lt = jnp.transpose(logits)
        row = jax.lax.broadcasted_iota(jnp.int32, (_E, _BT), 0)
        cnt = jnp.zeros((_E, _BT), jnp.float32)
        for e2 in range(_E):
            l2 = lt[e2:e2 + 1, :]
            beats = (l2 > lt) | ((l2 == lt) & (e2 < row))
            cnt += beats.astype(jnp.float32)
        sel = cnt < float(_K)
        m = jnp.max(lt, axis=0, keepdims=True)
        z = jnp.where(sel, jnp.exp(lt - m), 0.0)
        gt = z / jnp.sum(z, axis=0, keepdims=True)
        g_ref[...] = jnp.transpose(gt)

    @pl.when(e < _E // _EP)
    def _():
        col = jax.lax.broadcasted_iota(jnp.int32, (_BT, _E), 1)
        for j in range(_EP):
            h = jnp.dot(x_ref[...].astype(jnp.bfloat16),
                        w1_ref[j].astype(jnp.bfloat16),
                        preferred_element_type=jnp.float32)
            h = jnp.maximum(h + b1_ref[j], 0.0)
            o = jnp.dot(h.astype(jnp.bfloat16),
                        w2_ref[j].astype(jnp.bfloat16),
                        preferred_element_type=jnp.float32)
            o = o + b2_ref[j]
            ge = jnp.sum(jnp.where(col == e * _EP + j, g_ref[...], 0.0),
                         axis=1, keepdims=True)
            if j == 0:
                c_ref[...] = ge * o
            else:
                c_ref[...] += ge * o


def _wclamp(i, e):
    return (jnp.minimum(e, _E // _EP - 1), 0, 0)


def _moe_block(x, wg, w1, b1, w2, b2):
    return pl.pallas_call(
        _moe_body,
        grid=(_T // _BT, _E // _EP + 1),
        in_specs=[
            pl.BlockSpec((_BT, _D), lambda i, e: (i, 0)),
            pl.BlockSpec((_D, _E), lambda i, e: (0, 0)),
            pl.BlockSpec((_EP, _D, _F), _wclamp),
            pl.BlockSpec((_EP, 1, _F), _wclamp),
            pl.BlockSpec((_EP, _F, _D), _wclamp),
            pl.BlockSpec((_EP, 1, _D), _wclamp),
        ],
        out_specs=pl.BlockSpec((_BT, _D), lambda i, e: (i, 0)),
        out_shape=jax.ShapeDtypeStruct((_T, _D), jnp.float32),
        scratch_shapes=[pltpu.VMEM((_BT, _E), jnp.float32),
                        pltpu.VMEM((_BT, _D), jnp.float32)],
        compiler_params=pltpu.CompilerParams(
            dimension_semantics=("arbitrary", "arbitrary")),
    )(x, wg.astype(jnp.bfloat16), w1, b1.reshape(_E, 1, _F), w2,
      b2.reshape(_E, 1, _D))


@jax.jit
def kernel(x, Wg0, W1_0, b1_0, W2_0, b2_0, Wg1, W1_1, b1_1, W2_1, b2_1):
    h = _moe_block(x, Wg0, W1_0, b1_0, W2_0, b2_0)
    return _moe_block(h, Wg1, W1_1, b1_1, W2_1, b2_1)


# speedup vs baseline: 1.0818x; 1.0818x over previous
import jax
import jax.numpy as jnp
from jax.experimental import pallas as pl
from jax.experimental.pallas import tpu as pltpu

_T, _D, _E, _F, _K = 2048, 1024, 8, 1024, 4
_BT = 1024
_EP = 2


def _moe_body(x_ref, wg_ref, w1_ref, b1_ref, w2_ref, b2_ref, out_ref,
              g_ref, xb_ref):
    e = pl.program_id(1)

    @pl.when(e == 0)
    def _():
        xb_ref[...] = x_ref[...].astype(jnp.bfloat16)
        logits = jax.lax.dot_general(
            xb_ref[...], wg_ref[...], (((1,), (0,)), ((), ())),
            preferred_element_type=jnp.float32)
        lt = jnp.transpose(logits)
        row = jax.lax.broadcasted_iota(jnp.int32, (_E, _BT), 0)
        cnt = jnp.zeros((_E, _BT), jnp.float32)
        for e2 in range(_E):
            l2 = lt[e2:e2 + 1, :]
            beats = (l2 > lt) | ((l2 == lt) & (e2 < row))
            cnt += beats.astype(jnp.float32)
        sel = cnt < float(_K)
        m = jnp.max(lt, axis=0, keepdims=True)
        z = jnp.where(sel, jnp.exp(lt - m), 0.0)
        gt = z / jnp.sum(z, axis=0, keepdims=True)
        g_ref[...] = jnp.transpose(gt)

    col = jax.lax.broadcasted_iota(jnp.int32, (_BT, _E), 1)
    contrib = None
    for j in range(_EP):
        h = jnp.dot(xb_ref[...], w1_ref[j].astype(jnp.bfloat16),
                    preferred_element_type=jnp.float32)
        h = jnp.maximum(h + b1_ref[j], 0.0)
        o = jnp.dot(h.astype(jnp.bfloat16), w2_ref[j].astype(jnp.bfloat16),
                    preferred_element_type=jnp.float32)
        o = o + b2_ref[j]
        ge = jnp.sum(jnp.where(col == e * _EP + j, g_ref[...], 0.0),
                     axis=1, keepdims=True)
        contrib = ge * o if contrib is None else contrib + ge * o

    @pl.when(e == 0)
    def _():
        out_ref[...] = contrib

    @pl.when(e != 0)
    def _():
        out_ref[...] += contrib


def _moe_block(x, wg, w1, b1, w2, b2):
    return pl.pallas_call(
        _moe_body,
        grid=(_T // _BT, _E // _EP),
        in_specs=[
            pl.BlockSpec((_BT, _D), lambda i, e: (i, 0)),
            pl.BlockSpec((_D, _E), lambda i, e: (0, 0)),
            pl.BlockSpec((_EP, _D, _F), lambda i, e: (e, 0, 0)),
            pl.BlockSpec((_EP, 1, _F), lambda i, e: (e, 0, 0)),
            pl.BlockSpec((_EP, _F, _D), lambda i, e: (e, 0, 0)),
            pl.BlockSpec((_EP, 1, _D), lambda i, e: (e, 0, 0)),
        ],
        out_specs=pl.BlockSpec((_BT, _D), lambda i, e: (i, 0)),
        out_shape=jax.ShapeDtypeStruct((_T, _D), jnp.float32),
        scratch_shapes=[pltpu.VMEM((_BT, _E), jnp.float32),
                        pltpu.VMEM((_BT, _D), jnp.bfloat16)],
        compiler_params=pltpu.CompilerParams(
            dimension_semantics=("arbitrary", "arbitrary")),
    )(x, wg.astype(jnp.bfloat16), w1, b1.reshape(_E, 1, _F), w2,
      b2.reshape(_E, 1, _D))


@jax.jit
def kernel(x, Wg0, W1_0, b1_0, W2_0, b2_0, Wg1, W1_1, b1_1, W2_1, b2_1):
    h = _moe_block(x, Wg0, W1_0, b1_0, W2_0, b2_0)
    return _moe_block(h, Wg1, W1_1, b1_1, W2_1, b2_1)


# speedup vs baseline: 1.0862x; 1.0040x over previous
import jax
import jax.numpy as jnp
from jax.experimental import pallas as pl
from jax.experimental.pallas import tpu as pltpu

_T, _D, _E, _F, _K = 2048, 1024, 8, 1024, 4
_BT = 1024
_EP = 2


def _moe_body(x_ref, wg_ref, w1_ref, b1_ref, w2_ref, b2_ref, out_ref,
              g_ref, xb_ref):
    e = pl.program_id(1)

    @pl.when(e == 0)
    def _():
        xb_ref[...] = x_ref[...].astype(jnp.bfloat16)
        logits = jax.lax.dot_general(
            xb_ref[...], wg_ref[...], (((1,), (0,)), ((), ())),
            preferred_element_type=jnp.float32)
        lt = jnp.transpose(logits)
        row = jax.lax.broadcasted_iota(jnp.int32, (_E, _BT), 0)
        cnt = jnp.zeros((_E, _BT), jnp.float32)
        for e2 in range(_E):
            l2 = lt[e2:e2 + 1, :]
            beats = (l2 > lt) | ((l2 == lt) & (e2 < row))
            cnt += beats.astype(jnp.float32)
        sel = cnt < float(_K)
        m = jnp.max(lt, axis=0, keepdims=True)
        z = jnp.where(sel, jnp.exp(lt - m), 0.0)
        gt = z / jnp.sum(z, axis=0, keepdims=True)
        g_ref[...] = jnp.transpose(gt)

    col = jax.lax.broadcasted_iota(jnp.int32, (_BT, _E), 1)
    contrib = None
    for j in range(_EP):
        h = jnp.dot(xb_ref[...], w1_ref[j].astype(jnp.bfloat16),
                    preferred_element_type=jnp.float32)
        h = jnp.maximum(h + b1_ref[j], 0.0)
        o = jnp.dot(h.astype(jnp.bfloat16), w2_ref[j].astype(jnp.bfloat16),
                    preferred_element_type=jnp.float32)
        o = o + b2_ref[j]
        ge = jnp.sum(jnp.where(col == e * _EP + j, g_ref[...], 0.0),
                     axis=1, keepdims=True)
        contrib = ge * o if contrib is None else contrib + ge * o

    @pl.when(e == 0)
    def _():
        out_ref[...] = contrib

    @pl.when(e != 0)
    def _():
        out_ref[...] += contrib


def _moe_body2(x_ref, wg_ref, w1_ref, b1_ref, w2_ref, b2_ref, out_ref,
               g_ref, xb_ref):
    e = pl.program_id(1)

    @pl.when(e == 0)
    def _():
        xb_ref[...] = x_ref[...].astype(jnp.bfloat16)
        logits = jax.lax.dot_general(
            xb_ref[...], wg_ref[...], (((1,), (0,)), ((), ())),
            preferred_element_type=jnp.float32)
        lt = jnp.transpose(logits)
        row = jax.lax.broadcasted_iota(jnp.int32, (_E, _BT), 0)
        cnt = jnp.zeros((_E, _BT), jnp.float32)
        for e2 in range(_E):
            l2 = lt[e2:e2 + 1, :]
            beats = (l2 > lt) | ((l2 == lt) & (e2 < row))
            cnt += beats.astype(jnp.float32)
        sel = cnt < float(_K)
        m = jnp.max(lt, axis=0, keepdims=True)
        z = jnp.where(sel, jnp.exp(lt - m), 0.0)
        gt = z / jnp.sum(z, axis=0, keepdims=True)
        g_ref[...] = jnp.transpose(gt)

    col = jax.lax.broadcasted_iota(jnp.int32, (_BT, _E), 1)
    hp = []
    ges = []
    for j in range(_EP):
        h = jnp.dot(xb_ref[...], w1_ref[j].astype(jnp.bfloat16),
                    preferred_element_type=jnp.float32)
        h = jnp.maximum(h + b1_ref[j], 0.0)
        ge = jnp.sum(jnp.where(col == e * _EP + j, g_ref[...], 0.0),
                     axis=1, keepdims=True)
        hp.append((ge * h).astype(jnp.bfloat16))
        ges.append(ge)
    hp_cat = jnp.concatenate(hp, axis=1)
    w2_cat = w2_ref[...].reshape(_EP * _F, _D)
    o = jnp.dot(hp_cat, w2_cat.astype(jnp.bfloat16),
                preferred_element_type=jnp.float32)
    b2c = jnp.concatenate(
        [ges[j] * b2_ref[j] for j in range(_EP)], axis=0)
    contrib = o + b2c[0:1] + b2c[1:2] if False else o
    for j in range(_EP):
        contrib = contrib + ges[j] * b2_ref[j]

    @pl.when(e == 0)
    def _():
        out_ref[...] = contrib

    @pl.when(e != 0)
    def _():
        out_ref[...] += contrib


def _moe_block(x, wg, w1, b1, w2, b2, body=_moe_body):
    return pl.pallas_call(
        body,
        grid=(_T // _BT, _E // _EP),
        in_specs=[
            pl.BlockSpec((_BT, _D), lambda i, e: (i, 0)),
            pl.BlockSpec((_D, _E), lambda i, e: (0, 0)),
            pl.BlockSpec((_EP, _D, _F), lambda i, e: (e, 0, 0)),
            pl.BlockSpec((_EP, 1, _F), lambda i, e: (e, 0, 0)),
            pl.BlockSpec((_EP, _F, _D), lambda i, e: (e, 0, 0)),
            pl.BlockSpec((_EP, 1, _D), lambda i, e: (e, 0, 0)),
        ],
        out_specs=pl.BlockSpec((_BT, _D), lambda i, e: (i, 0)),
        out_shape=jax.ShapeDtypeStruct((_T, _D), jnp.float32),
        scratch_shapes=[pltpu.VMEM((_BT, _E), jnp.float32),
                        pltpu.VMEM((_BT, _D), jnp.bfloat16)],
        compiler_params=pltpu.CompilerParams(
            dimension_semantics=("arbitrary", "arbitrary")),
    )(x, wg.astype(jnp.bfloat16), w1, b1.reshape(_E, 1, _F), w2,
      b2.reshape(_E, 1, _D))


@jax.jit
def kernel(x, Wg0, W1_0, b1_0, W2_0, b2_0, Wg1, W1_1, b1_1, W2_1, b2_1):
    h = _moe_block(x, Wg0, W1_0, b1_0, W2_0, b2_0)
    return _moe_block(h, Wg1, W1_1, b1_1, W2_1, b2_1, body=_moe_body2)
